# baseline (speedup 1.0000x reference)
import functools

import jax
import jax.numpy as jnp
from jax import lax
from jax.experimental import pallas as pl
from jax.experimental.pallas import tpu as pltpu
from jax.experimental.pallas import tpu_sc as plsc

N = 10000
D = 128
E = 320000
NC = 2
NS = 16
NW = NC * NS
IR = 8
EROWS = 2560
E_PAD = EROWS * 128
ITERS = EROWS // (NW * IR)
RPW0 = 160
RPW1 = 160 - RPW0
N_PAD = 10112
R = 1000
GRID = N // R

_mesh = lambda: plsc.VectorSubcoreMesh(core_axis_name="c", subcore_axis_name="s")


def _sc_degree(dst2d):
    @functools.partial(
        pl.kernel,
        out_type=jax.ShapeDtypeStruct((NC * N,), jnp.float32),
        mesh=_mesh(),
        scratch_types=[
            pltpu.VMEM((IR, 128), jnp.int32),
            pltpu.VMEM((128,), jnp.float32),
            pltpu.VMEM((1280,), jnp.float32),
            pltpu.VMEM((2000,), jnp.float32),
            pltpu.VMEM_SHARED((10240,), jnp.float32),
            pltpu.SemaphoreType.DMA,
        ],
    )
    def k(dst_hbm, out_hbm, di_v, ones_v, zw_v, wb_v, deg_sh, sem):
        c = lax.axis_index("c")
        s = lax.axis_index("s")
        wid = s * NC + c
        for j in range(8):
            ones_v[pl.ds(j * 16, 16)] = jnp.ones((16,), jnp.float32)
        for j in range(80):
            zw_v[pl.ds(j * 16, 16)] = jnp.zeros((16,), jnp.float32)
        @pl.when(s < 8)
        def _():
            pltpu.sync_copy(zw_v, deg_sh.at[pl.ds(s * 1280, 1280)])
        plsc.subcore_barrier()

        def body(i, carry):
            row = (wid * ITERS + i) * IR
            pltpu.sync_copy(dst_hbm.at[pl.ds(row, IR)], di_v)
            for j in range(IR):
                pltpu.sync_copy(ones_v, deg_sh.at[di_v.at[j]], add=True)
            return carry

        lax.fori_loop(0, ITERS, body, 0)
        plsc.subcore_barrier()
        @pl.when(s < 5)
        def _():
            pltpu.sync_copy(deg_sh.at[pl.ds(s * 2000, 2000)], wb_v)
            pltpu.sync_copy(wb_v, out_hbm.at[pl.ds(c * N + s * 2000, 2000)])

    return k(dst2d)


def _sc_aggregate(table, src2d, dst2d, zeros256):
    @functools.partial(
        pl.kernel,
        out_type=jax.ShapeDtypeStruct((NC, N, D), jnp.float32),
        mesh=_mesh(),
        scratch_types=[
            pltpu.VMEM((16, 128), jnp.int32),
            pltpu.VMEM((16, 128), jnp.int32),
            pltpu.VMEM((256, D), jnp.float32),
            pltpu.VMEM_SHARED((N_PAD, D), jnp.float32),
            pltpu.SemaphoreType.DMA,
            pltpu.SemaphoreType.DMA,
        ],
    )
    def k(t_hbm, src_hbm, dst_hbm, z_hbm, out_hbm, si_v, di_v, rows_v,
          acc_sh, gsem, ssem):
        c = lax.axis_index("c")
        s = lax.axis_index("s")
        wid = s * NC + c
        pltpu.sync_copy(z_hbm, rows_v)
        zbase = s * 632
        for j in range(2):
            pltpu.sync_copy(rows_v, acc_sh.at[pl.ds(zbase + j * 256, 256)])
        pltpu.sync_copy(rows_v.at[pl.ds(0, 120)],
                        acc_sh.at[pl.ds(zbase + 512, 120)])
        plsc.subcore_barrier()

        base = jnp.where(c == 0, s * RPW0, 16 * RPW0 + s * RPW1)
        my_iters = jnp.where(c == 0, RPW0 // 16, RPW1 // 16)

        def body(blk, carry):
            row = base + blk * 16
            pltpu.sync_copy(src_hbm.at[pl.ds(row, 16)], si_v)
            pltpu.sync_copy(dst_hbm.at[pl.ds(row, 16)], di_v)
            NU = 16
            g = [None] * NU
            sd = [None] * NU
            g[0] = pltpu.async_copy(t_hbm.at[si_v.at[0]],
                                    rows_v.at[pl.ds(0, 128)], gsem)
            for j in range(NU):
                g[j].wait()
                if j < NU - 1:
                    if j >= 1:
                        sd[j - 1].wait()
                    g[j + 1] = pltpu.async_copy(
                        t_hbm.at[si_v.at[j + 1]],
                        rows_v.at[pl.ds(((j + 1) % 2) * 128, 128)], gsem)
                sd[j] = pltpu.async_copy(
                    rows_v.at[pl.ds((j % 2) * 128, 128)],
                    acc_sh.at[di_v.at[j]], ssem, add=True)
            sd[NU - 2].wait()
            sd[NU - 1].wait()
            return carry

        def guarded(blk, carry):
            @pl.when(blk < my_iters)
            def _():
                body(blk, 0)
            return carry

        lax.fori_loop(0, max(RPW0, RPW1) // 16, guarded, 0)
        plsc.subcore_barrier()
        wbase = s * 632

        @pl.when(s < 15)
        def _():
            for j in range(2):
                pltpu.sync_copy(acc_sh.at[pl.ds(wbase + j * 256, 256)], rows_v)
                pltpu.sync_copy(rows_v,
                                out_hbm.at[c, pl.ds(wbase + j * 256, 256)])
            pltpu.sync_copy(acc_sh.at[pl.ds(wbase + 512, 120)],
                            rows_v.at[pl.ds(0, 120)])
            pltpu.sync_copy(rows_v.at[pl.ds(0, 120)],
                            out_hbm.at[c, pl.ds(wbase + 512, 120)])

        @pl.when(s == 15)
        def _():
            for j in range(2):
                pltpu.sync_copy(acc_sh.at[pl.ds(9480 + j * 256, 256)], rows_v)
                pltpu.sync_copy(rows_v,
                                out_hbm.at[c, pl.ds(9480 + j * 256, 256)])
            pltpu.sync_copy(acc_sh.at[pl.ds(9992, 8)], rows_v.at[pl.ds(0, 8)])
            pltpu.sync_copy(rows_v.at[pl.ds(0, 8)],
                            out_hbm.at[c, pl.ds(9992, 8)])

    return k(table, src2d, dst2d, zeros256)


def _row_spec(block_rows, cols):
    return pl.BlockSpec((block_rows, cols), lambda i: (i, 0))


def _fixed_spec(shape):
    nd = len(shape)
    return pl.BlockSpec(shape, lambda i: (0,) * nd)


def _tc_mm(x, W1):
    def body(x_ref, w_ref, out_ref):
        out_ref[...] = jnp.dot(x_ref[...], w_ref[...],
                               preferred_element_type=jnp.float32)

    return pl.pallas_call(
        body,
        grid=(GRID,),
        in_specs=[_row_spec(R, D), _fixed_spec((D, D))],
        out_specs=_row_spec(R, D),
        out_shape=jax.ShapeDtypeStruct((N, D), jnp.float32),
    )(x, W1)


def _tc_first(degpair, mm):
    def body(deg_ref, mm_ref, t_ref, dis_ref):
        deg = deg_ref[...]
        dis = lax.rsqrt(deg[:, 0:1] + deg[:, 1:2] + 1.0)
        dis_ref[...] = dis
        t_ref[...] = dis * mm_ref[...]

    return pl.pallas_call(
        body,
        grid=(GRID,),
        in_specs=[_row_spec(R, 2), _row_spec(R, D)],
        out_specs=[_row_spec(R, D), _row_spec(R, 1)],
        out_shape=[jax.ShapeDtypeStruct((N, D), jnp.float32),
                   jax.ShapeDtypeStruct((N, 1), jnp.float32)],
    )(degpair, mm)


def _tc_mid(dis, S, t_prev, b_prev, W_next):
    def body(dis_ref, sa_ref, sb_ref, t_ref, b_ref, w_ref, out_ref):
        dis = dis_ref[...]
        h = dis * (sa_ref[0] + sb_ref[0] + t_ref[...]) + b_ref[...]
        h = jnp.maximum(h, 0.0)
        out_ref[...] = dis * jnp.dot(h, w_ref[...],
                                     preferred_element_type=jnp.float32)

    return pl.pallas_call(
        body,
        grid=(GRID,),
        in_specs=[
            _row_spec(R, 1),
            pl.BlockSpec((1, R, D), lambda i: (0, i, 0)),
            pl.BlockSpec((1, R, D), lambda i: (1, i, 0)),
            _row_spec(R, D),
            _fixed_spec((1, D)),
            _fixed_spec((D, D)),
        ],
        out_specs=_row_spec(R, D),
        out_shape=jax.ShapeDtypeStruct((N, D), jnp.float32),
    )(dis, S, S, t_prev, b_prev, W_next)


def _tc_last(dis, S, t_prev, b_prev):
    def body(dis_ref, sa_ref, sb_ref, t_ref, b_ref, out_ref):
        out_ref[...] = (dis_ref[...] * (sa_ref[0] + sb_ref[0] + t_ref[...])
                        + b_ref[...])

    return pl.pallas_call(
        body,
        grid=(GRID,),
        in_specs=[
            _row_spec(R, 1),
            pl.BlockSpec((1, R, D), lambda i: (0, i, 0)),
            pl.BlockSpec((1, R, D), lambda i: (1, i, 0)),
            _row_spec(R, D),
            _fixed_spec((1, D)),
        ],
        out_specs=_row_spec(R, D),
        out_shape=jax.ShapeDtypeStruct((N, D), jnp.float32),
    )(dis, S, S, t_prev, b_prev)


def kernel(x, edge_index, W1, b1, W2, b2, W3, b3, W4, b4):
    pad = E_PAD - E
    src2d = jnp.concatenate(
        [edge_index[0], jnp.zeros((pad,), jnp.int32)]).reshape(EROWS, 128)
    dump = N + (jnp.arange(pad, dtype=jnp.int32) % (N_PAD - N))
    dst2d = jnp.concatenate([edge_index[1], dump]).reshape(EROWS, 128)
    zeros256 = jnp.zeros((256, D), jnp.float32)
    degp = _sc_degree(dst2d)
    mm1 = _tc_mm(x, W1)
    degpair = degp.reshape(NC, N).T
    t1, dis = _tc_first(degpair, mm1)
    s1 = _sc_aggregate(t1, src2d, dst2d, zeros256)
    t2 = _tc_mid(dis, s1, t1, b1.reshape(1, D), W2)
    s2 = _sc_aggregate(t2, src2d, dst2d, zeros256)
    t3 = _tc_mid(dis, s2, t2, b2.reshape(1, D), W3)
    s3 = _sc_aggregate(t3, src2d, dst2d, zeros256)
    t4 = _tc_mid(dis, s3, t3, b3.reshape(1, D), W4)
    s4 = _sc_aggregate(t4, src2d, dst2d, zeros256)
    return _tc_last(dis, s4, t4, b4.reshape(1, D))


# speedup vs baseline: 3.5666x; 3.5666x over previous
import functools

import jax
import jax.numpy as jnp
from jax import lax
from jax.experimental import pallas as pl
from jax.experimental.pallas import tpu as pltpu
from jax.experimental.pallas import tpu_sc as plsc

N = 10000
D = 128
E = 320000
NC = 2
NS = 16
NW = NC * NS
IR = 8
EROWS = 2504
E_PAD = EROWS * 128
TOTB = EROWS // IR
BPW = -(-TOTB // NW)
N_PAD = 10112
R = 1000
GRID = N // R

_mesh = lambda: plsc.VectorSubcoreMesh(core_axis_name="c", subcore_axis_name="s")


def _sc_degree(dst2d):
    @functools.partial(
        pl.kernel,
        out_type=jax.ShapeDtypeStruct((NC * N,), jnp.float32),
        mesh=_mesh(),
        scratch_types=[
            pltpu.VMEM((IR, 128), jnp.int32),
            pltpu.VMEM((128,), jnp.float32),
            pltpu.VMEM((1280,), jnp.float32),
            pltpu.VMEM((2000,), jnp.float32),
            pltpu.VMEM_SHARED((10240,), jnp.float32),
            pltpu.SemaphoreType.DMA,
        ],
    )
    def k(dst_hbm, out_hbm, di_v, ones_v, zw_v, wb_v, deg_sh, sem):
        c = lax.axis_index("c")
        s = lax.axis_index("s")
        wid = s * NC + c
        for j in range(8):
            ones_v[pl.ds(j * 16, 16)] = jnp.ones((16,), jnp.float32)
        for j in range(80):
            zw_v[pl.ds(j * 16, 16)] = jnp.zeros((16,), jnp.float32)
        @pl.when(s < 8)
        def _():
            pltpu.sync_copy(zw_v, deg_sh.at[pl.ds(s * 1280, 1280)])
        plsc.subcore_barrier()

        nblk = jnp.where(wid < TOTB - NW * (BPW - 1), BPW, BPW - 1)
        base_blk = BPW * wid - jnp.maximum(0, wid - (TOTB - NW * (BPW - 1)))

        def body(i, carry):
            @pl.when(i < nblk)
            def _():
                row = (base_blk + i) * IR
                pltpu.sync_copy(dst_hbm.at[pl.ds(row, IR)], di_v)
                for j in range(IR):
                    pltpu.sync_copy(ones_v, deg_sh.at[di_v.at[j]], add=True)
            return carry

        lax.fori_loop(0, BPW, body, 0)
        plsc.subcore_barrier()
        @pl.when(s < 5)
        def _():
            pltpu.sync_copy(deg_sh.at[pl.ds(s * 2000, 2000)], wb_v)
            pltpu.sync_copy(wb_v, out_hbm.at[pl.ds(c * N + s * 2000, 2000)])

    return k(dst2d)


def _sc_aggregate(table, src2d, dst2d, zeros256):
    @functools.partial(
        pl.kernel,
        out_type=jax.ShapeDtypeStruct((NC, N, D), jnp.float32),
        mesh=_mesh(),
        scratch_types=[
            pltpu.VMEM((IR, 128), jnp.int32),
            pltpu.VMEM((IR, 128), jnp.int32),
            pltpu.VMEM((256, D), jnp.float32),
            pltpu.VMEM_SHARED((N_PAD, D), jnp.float32),
            pltpu.SemaphoreType.DMA,
            pltpu.SemaphoreType.DMA,
        ],
    )
    def k(t_hbm, src_hbm, dst_hbm, z_hbm, out_hbm, si_v, di_v, rows_v,
          acc_sh, gsem, ssem):
        c = lax.axis_index("c")
        s = lax.axis_index("s")
        wid = s * NC + c
        pltpu.sync_copy(z_hbm, rows_v)
        zbase = s * 632
        for j in range(2):
            pltpu.sync_copy(rows_v, acc_sh.at[pl.ds(zbase + j * 256, 256)])
        pltpu.sync_copy(rows_v.at[pl.ds(0, 120)],
                        acc_sh.at[pl.ds(zbase + 512, 120)])
        plsc.subcore_barrier()

        nblk = jnp.where(wid < TOTB - NW * (BPW - 1), BPW, BPW - 1)
        base_blk = BPW * wid - jnp.maximum(0, wid - (TOTB - NW * (BPW - 1)))

        def body(blk, carry):
            row = (base_blk + blk) * IR
            pltpu.sync_copy(src_hbm.at[pl.ds(row, IR)], si_v)
            pltpu.sync_copy(dst_hbm.at[pl.ds(row, IR)], di_v)
            NU = IR
            g = [None] * NU
            sd = [None] * NU
            g[0] = pltpu.async_copy(t_hbm.at[si_v.at[0]],
                                    rows_v.at[pl.ds(0, 128)], gsem)
            for j in range(NU):
                g[j].wait()
                if j < NU - 1:
                    if j >= 1:
                        sd[j - 1].wait()
                    g[j + 1] = pltpu.async_copy(
                        t_hbm.at[si_v.at[j + 1]],
                        rows_v.at[pl.ds(((j + 1) % 2) * 128, 128)], gsem)
                sd[j] = pltpu.async_copy(
                    rows_v.at[pl.ds((j % 2) * 128, 128)],
                    acc_sh.at[di_v.at[j]], ssem, add=True)
            sd[NU - 2].wait()
            sd[NU - 1].wait()
            return carry

        def guarded(blk, carry):
            @pl.when(blk < nblk)
            def _():
                body(blk, 0)
            return carry

        lax.fori_loop(0, BPW, guarded, 0)
        plsc.subcore_barrier()
        wbase = s * 632

        @pl.when(s < 15)
        def _():
            for j in range(2):
                pltpu.sync_copy(acc_sh.at[pl.ds(wbase + j * 256, 256)], rows_v)
                pltpu.sync_copy(rows_v,
                                out_hbm.at[c, pl.ds(wbase + j * 256, 256)])
            pltpu.sync_copy(acc_sh.at[pl.ds(wbase + 512, 120)],
                            rows_v.at[pl.ds(0, 120)])
            pltpu.sync_copy(rows_v.at[pl.ds(0, 120)],
                            out_hbm.at[c, pl.ds(wbase + 512, 120)])

        @pl.when(s == 15)
        def _():
            for j in range(2):
                pltpu.sync_copy(acc_sh.at[pl.ds(9480 + j * 256, 256)], rows_v)
                pltpu.sync_copy(rows_v,
                                out_hbm.at[c, pl.ds(9480 + j * 256, 256)])
            pltpu.sync_copy(acc_sh.at[pl.ds(9992, 8)], rows_v.at[pl.ds(0, 8)])
            pltpu.sync_copy(rows_v.at[pl.ds(0, 8)],
                            out_hbm.at[c, pl.ds(9992, 8)])

    return k(table, src2d, dst2d, zeros256)


def _row_spec(block_rows, cols):
    return pl.BlockSpec((block_rows, cols), lambda i: (i, 0))


def _fixed_spec(shape):
    nd = len(shape)
    return pl.BlockSpec(shape, lambda i: (0,) * nd)


def _tc_mm(x, W1):
    def body(x_ref, w_ref, out_ref):
        out_ref[...] = jnp.dot(x_ref[...], w_ref[...],
                               preferred_element_type=jnp.float32)

    return pl.pallas_call(
        body,
        grid=(GRID,),
        in_specs=[_row_spec(R, D), _fixed_spec((D, D))],
        out_specs=_row_spec(R, D),
        out_shape=jax.ShapeDtypeStruct((N, D), jnp.float32),
    )(x, W1)


def _tc_first(degpair, mm):
    def body(deg_ref, mm_ref, t_ref, dis_ref):
        deg = deg_ref[...]
        dis = lax.rsqrt(deg[:, 0:1] + deg[:, 1:2] + 1.0)
        dis_ref[...] = dis
        t_ref[...] = dis * mm_ref[...]

    return pl.pallas_call(
        body,
        grid=(GRID,),
        in_specs=[_row_spec(R, 2), _row_spec(R, D)],
        out_specs=[_row_spec(R, D), _row_spec(R, 1)],
        out_shape=[jax.ShapeDtypeStruct((N, D), jnp.float32),
                   jax.ShapeDtypeStruct((N, 1), jnp.float32)],
    )(degpair, mm)


def _tc_mid(dis, S, t_prev, b_prev, W_next):
    def body(dis_ref, sa_ref, sb_ref, t_ref, b_ref, w_ref, out_ref):
        dis = dis_ref[...]
        h = dis * (sa_ref[0] + sb_ref[0] + t_ref[...]) + b_ref[...]
        h = jnp.maximum(h, 0.0)
        out_ref[...] = dis * jnp.dot(h, w_ref[...],
                                     preferred_element_type=jnp.float32)

    return pl.pallas_call(
        body,
        grid=(GRID,),
        in_specs=[
            _row_spec(R, 1),
            pl.BlockSpec((1, R, D), lambda i: (0, i, 0)),
            pl.BlockSpec((1, R, D), lambda i: (1, i, 0)),
            _row_spec(R, D),
            _fixed_spec((1, D)),
            _fixed_spec((D, D)),
        ],
        out_specs=_row_spec(R, D),
        out_shape=jax.ShapeDtypeStruct((N, D), jnp.float32),
    )(dis, S, S, t_prev, b_prev, W_next)


def _tc_last(dis, S, t_prev, b_prev):
    def body(dis_ref, sa_ref, sb_ref, t_ref, b_ref, out_ref):
        out_ref[...] = (dis_ref[...] * (sa_ref[0] + sb_ref[0] + t_ref[...])
                        + b_ref[...])

    return pl.pallas_call(
        body,
        grid=(GRID,),
        in_specs=[
            _row_spec(R, 1),
            pl.BlockSpec((1, R, D), lambda i: (0, i, 0)),
            pl.BlockSpec((1, R, D), lambda i: (1, i, 0)),
            _row_spec(R, D),
            _fixed_spec((1, D)),
        ],
        out_specs=_row_spec(R, D),
        out_shape=jax.ShapeDtypeStruct((N, D), jnp.float32),
    )(dis, S, S, t_prev, b_prev)


def kernel(x, edge_index, W1, b1, W2, b2, W3, b3, W4, b4):
    pad = E_PAD - E
    pad_src = jnp.arange(pad, dtype=jnp.int32) % N
    src2d = jnp.concatenate([edge_index[0], pad_src]).reshape(EROWS, 128)
    dump = N + (jnp.arange(pad, dtype=jnp.int32) % (N_PAD - N))
    dst2d = jnp.concatenate([edge_index[1], dump]).reshape(EROWS, 128)
    zeros256 = jnp.zeros((256, D), jnp.float32)
    degp = _sc_degree(dst2d)
    mm1 = _tc_mm(x, W1)
    degpair = degp.reshape(NC, N).T
    t1, dis = _tc_first(degpair, mm1)
    s1 = _sc_aggregate(t1, src2d, dst2d, zeros256)
    t2 = _tc_mid(dis, s1, t1, b1.reshape(1, D), W2)
    s2 = _sc_aggregate(t2, src2d, dst2d, zeros256)
    t3 = _tc_mid(dis, s2, t2, b2.reshape(1, D), W3)
    s3 = _sc_aggregate(t3, src2d, dst2d, zeros256)
    t4 = _tc_mid(dis, s3, t3, b3.reshape(1, D), W4)
    s4 = _sc_aggregate(t4, src2d, dst2d, zeros256)
    return _tc_last(dis, s4, t4, b4.reshape(1, D))


# speedup vs baseline: 3.5990x; 1.0091x over previous
import functools

import jax
import jax.numpy as jnp
from jax import lax
from jax.experimental import pallas as pl
from jax.experimental.pallas import tpu as pltpu
from jax.experimental.pallas import tpu_sc as plsc

N = 10000
D = 128
E = 320000
NC = 2
NS = 16
NW = NC * NS
IR = 8
EROWS = 2504
E_PAD = EROWS * 128
TOTB = EROWS // IR
BPW = -(-TOTB // NW)
N_PAD = 10112
R = 1000
GRID = N // R

_mesh = lambda: plsc.VectorSubcoreMesh(core_axis_name="c", subcore_axis_name="s")


def _sc_degree(dst2d):
    @functools.partial(
        pl.kernel,
        out_type=jax.ShapeDtypeStruct((NC * N,), jnp.float32),
        mesh=_mesh(),
        scratch_types=[
            pltpu.VMEM((IR, 128), jnp.int32),
            pltpu.VMEM((128,), jnp.float32),
            pltpu.VMEM((1280,), jnp.float32),
            pltpu.VMEM((2000,), jnp.float32),
            pltpu.VMEM_SHARED((10240,), jnp.float32),
            pltpu.SemaphoreType.DMA,
        ],
    )
    def k(dst_hbm, out_hbm, di_v, ones_v, zw_v, wb_v, deg_sh, sem):
        c = lax.axis_index("c")
        s = lax.axis_index("s")
        wid = s * NC + c
        for j in range(8):
            ones_v[pl.ds(j * 16, 16)] = jnp.ones((16,), jnp.float32)
        for j in range(80):
            zw_v[pl.ds(j * 16, 16)] = jnp.zeros((16,), jnp.float32)
        @pl.when(s < 8)
        def _():
            pltpu.sync_copy(zw_v, deg_sh.at[pl.ds(s * 1280, 1280)])
        plsc.subcore_barrier()

        nblk = jnp.where(wid < TOTB - NW * (BPW - 1), BPW, BPW - 1)
        base_blk = BPW * wid - jnp.maximum(0, wid - (TOTB - NW * (BPW - 1)))

        def body(i, carry):
            @pl.when(i < nblk)
            def _():
                row = (base_blk + i) * IR
                pltpu.sync_copy(dst_hbm.at[pl.ds(row, IR)], di_v)
                for j in range(IR):
                    pltpu.sync_copy(ones_v, deg_sh.at[di_v.at[j]], add=True)
            return carry

        lax.fori_loop(0, BPW, body, 0)
        plsc.subcore_barrier()
        @pl.when(s < 5)
        def _():
            pltpu.sync_copy(deg_sh.at[pl.ds(s * 2000, 2000)], wb_v)
            pltpu.sync_copy(wb_v, out_hbm.at[pl.ds(c * N + s * 2000, 2000)])

    return k(dst2d)


def _sc_aggregate(table, src2d, dst2d, zeros256):
    @functools.partial(
        pl.kernel,
        out_type=jax.ShapeDtypeStruct((NC, N, D), jnp.float32),
        mesh=_mesh(),
        scratch_types=[
            pltpu.VMEM((IR, 128), jnp.int32),
            pltpu.VMEM((IR, 128), jnp.int32),
            pltpu.VMEM((256, D), jnp.float32),
            pltpu.VMEM_SHARED((N_PAD, D), jnp.float32),
            pltpu.SemaphoreType.DMA,
            pltpu.SemaphoreType.DMA,
        ],
    )
    def k(t_hbm, src_hbm, dst_hbm, z_hbm, out_hbm, si_v, di_v, rows_v,
          acc_sh, gsem, ssem):
        c = lax.axis_index("c")
        s = lax.axis_index("s")
        wid = s * NC + c
        pltpu.sync_copy(z_hbm, rows_v)
        zbase = s * 632
        zcps = [pltpu.async_copy(rows_v, acc_sh.at[pl.ds(zbase + j * 256, 256)],
                                 gsem) for j in range(2)]
        zcps.append(pltpu.async_copy(rows_v.at[pl.ds(0, 120)],
                                     acc_sh.at[pl.ds(zbase + 512, 120)], gsem))
        for cp in zcps:
            cp.wait()
        plsc.subcore_barrier()

        nblk = jnp.where(wid < TOTB - NW * (BPW - 1), BPW, BPW - 1)
        base_blk = BPW * wid - jnp.maximum(0, wid - (TOTB - NW * (BPW - 1)))

        def body(blk, carry):
            row = (base_blk + blk) * IR
            pltpu.sync_copy(src_hbm.at[pl.ds(row, IR)], si_v)
            pltpu.sync_copy(dst_hbm.at[pl.ds(row, IR)], di_v)
            NU = IR
            g = [None] * NU
            sd = [None] * NU
            g[0] = pltpu.async_copy(t_hbm.at[si_v.at[0]],
                                    rows_v.at[pl.ds(0, 128)], gsem)
            for j in range(NU):
                g[j].wait()
                if j < NU - 1:
                    if j >= 1:
                        sd[j - 1].wait()
                    g[j + 1] = pltpu.async_copy(
                        t_hbm.at[si_v.at[j + 1]],
                        rows_v.at[pl.ds(((j + 1) % 2) * 128, 128)], gsem)
                sd[j] = pltpu.async_copy(
                    rows_v.at[pl.ds((j % 2) * 128, 128)],
                    acc_sh.at[di_v.at[j]], ssem, add=True)
            sd[NU - 2].wait()
            sd[NU - 1].wait()
            return carry

        def guarded(blk, carry):
            @pl.when(blk < nblk)
            def _():
                body(blk, 0)
            return carry

        lax.fori_loop(0, BPW, guarded, 0)
        plsc.subcore_barrier()
        wbase = s * 632

        def writeout(base_row, sizes):
            offs = []
            o = 0
            for sz in sizes:
                offs.append(o)
                o += sz
            a = [None] * len(sizes)
            b = [None] * len(sizes)
            a[0] = pltpu.async_copy(
                acc_sh.at[pl.ds(base_row + offs[0], sizes[0])],
                rows_v.at[pl.ds(0, sizes[0])], gsem)
            for j, sz in enumerate(sizes):
                h = (j % 2) * 128
                a[j].wait()
                if j + 1 < len(sizes):
                    if j >= 1:
                        b[j - 1].wait()
                    hn = ((j + 1) % 2) * 128
                    a[j + 1] = pltpu.async_copy(
                        acc_sh.at[pl.ds(base_row + offs[j + 1], sizes[j + 1])],
                        rows_v.at[pl.ds(hn, sizes[j + 1])], gsem)
                b[j] = pltpu.async_copy(
                    rows_v.at[pl.ds(h, sz)],
                    out_hbm.at[c, pl.ds(base_row + offs[j], sz)], ssem)
            for j in range(max(0, len(sizes) - 2), len(sizes)):
                b[j].wait()

        @pl.when(s < 15)
        def _():
            writeout(wbase, [128, 128, 128, 128, 120])

        @pl.when(s == 15)
        def _():
            writeout(9480, [128, 128, 128, 128, 8])

    return k(table, src2d, dst2d, zeros256)


def _row_spec(block_rows, cols):
    return pl.BlockSpec((block_rows, cols), lambda i: (i, 0))


def _fixed_spec(shape):
    nd = len(shape)
    return pl.BlockSpec(shape, lambda i: (0,) * nd)


def _tc_mm(x, W1):
    def body(x_ref, w_ref, out_ref):
        out_ref[...] = jnp.dot(x_ref[...], w_ref[...],
                               preferred_element_type=jnp.float32)

    return pl.pallas_call(
        body,
        grid=(GRID,),
        in_specs=[_row_spec(R, D), _fixed_spec((D, D))],
        out_specs=_row_spec(R, D),
        out_shape=jax.ShapeDtypeStruct((N, D), jnp.float32),
    )(x, W1)


def _tc_first(degpair, mm):
    def body(deg_ref, mm_ref, t_ref, dis_ref):
        deg = deg_ref[...]
        dis = lax.rsqrt(deg[:, 0:1] + deg[:, 1:2] + 1.0)
        dis_ref[...] = dis
        t_ref[...] = dis * mm_ref[...]

    return pl.pallas_call(
        body,
        grid=(GRID,),
        in_specs=[_row_spec(R, 2), _row_spec(R, D)],
        out_specs=[_row_spec(R, D), _row_spec(R, 1)],
        out_shape=[jax.ShapeDtypeStruct((N, D), jnp.float32),
                   jax.ShapeDtypeStruct((N, 1), jnp.float32)],
    )(degpair, mm)


def _tc_mid(dis, S, t_prev, b_prev, W_next):
    def body(dis_ref, sa_ref, sb_ref, t_ref, b_ref, w_ref, out_ref):
        dis = dis_ref[...]
        h = dis * (sa_ref[0] + sb_ref[0] + t_ref[...]) + b_ref[...]
        h = jnp.maximum(h, 0.0)
        out_ref[...] = dis * jnp.dot(h, w_ref[...],
                                     preferred_element_type=jnp.float32)

    return pl.pallas_call(
        body,
        grid=(GRID,),
        in_specs=[
            _row_spec(R, 1),
            pl.BlockSpec((1, R, D), lambda i: (0, i, 0)),
            pl.BlockSpec((1, R, D), lambda i: (1, i, 0)),
            _row_spec(R, D),
            _fixed_spec((1, D)),
            _fixed_spec((D, D)),
        ],
        out_specs=_row_spec(R, D),
        out_shape=jax.ShapeDtypeStruct((N, D), jnp.float32),
    )(dis, S, S, t_prev, b_prev, W_next)


def _tc_last(dis, S, t_prev, b_prev):
    def body(dis_ref, sa_ref, sb_ref, t_ref, b_ref, out_ref):
        out_ref[...] = (dis_ref[...] * (sa_ref[0] + sb_ref[0] + t_ref[...])
                        + b_ref[...])

    return pl.pallas_call(
        body,
        grid=(GRID,),
        in_specs=[
            _row_spec(R, 1),
            pl.BlockSpec((1, R, D), lambda i: (0, i, 0)),
            pl.BlockSpec((1, R, D), lambda i: (1, i, 0)),
            _row_spec(R, D),
            _fixed_spec((1, D)),
        ],
        out_specs=_row_spec(R, D),
        out_shape=jax.ShapeDtypeStruct((N, D), jnp.float32),
    )(dis, S, S, t_prev, b_prev)


def kernel(x, edge_index, W1, b1, W2, b2, W3, b3, W4, b4):
    pad = E_PAD - E
    pad_src = jnp.arange(pad, dtype=jnp.int32) % N
    src2d = jnp.concatenate([edge_index[0], pad_src]).reshape(EROWS, 128)
    dump = N + (jnp.arange(pad, dtype=jnp.int32) % (N_PAD - N))
    dst2d = jnp.concatenate([edge_index[1], dump]).reshape(EROWS, 128)
    zeros256 = jnp.zeros((256, D), jnp.float32)
    degp = _sc_degree(dst2d)
    mm1 = _tc_mm(x, W1)
    degpair = degp.reshape(NC, N).T
    t1, dis = _tc_first(degpair, mm1)
    s1 = _sc_aggregate(t1, src2d, dst2d, zeros256)
    t2 = _tc_mid(dis, s1, t1, b1.reshape(1, D), W2)
    s2 = _sc_aggregate(t2, src2d, dst2d, zeros256)
    t3 = _tc_mid(dis, s2, t2, b2.reshape(1, D), W3)
    s3 = _sc_aggregate(t3, src2d, dst2d, zeros256)
    t4 = _tc_mid(dis, s3, t3, b3.reshape(1, D), W4)
    s4 = _sc_aggregate(t4, src2d, dst2d, zeros256)
    return _tc_last(dis, s4, t4, b4.reshape(1, D))


# speedup vs baseline: 3.7686x; 1.0471x over previous
import functools

import jax
import jax.numpy as jnp
from jax import lax
from jax.experimental import pallas as pl
from jax.experimental.pallas import tpu as pltpu
from jax.experimental.pallas import tpu_sc as plsc

N = 10000
D = 128
E = 320000
NC = 2
NS = 16
NW = NC * NS
IR = 16
EROWS = 2512
E_PAD = EROWS * 128
TOTB = EROWS // IR
BPW = -(-TOTB // NW)
N_PAD = 10112
R = 1000
GRID = N // R

_mesh = lambda: plsc.VectorSubcoreMesh(core_axis_name="c", subcore_axis_name="s")


def _sc_degree(dst2d):
    @functools.partial(
        pl.kernel,
        out_type=jax.ShapeDtypeStruct((NC * N,), jnp.float32),
        mesh=_mesh(),
        scratch_types=[
            pltpu.VMEM((IR, 128), jnp.int32),
            pltpu.VMEM((128,), jnp.float32),
            pltpu.VMEM((1280,), jnp.float32),
            pltpu.VMEM((2000,), jnp.float32),
            pltpu.VMEM_SHARED((10240,), jnp.float32),
            pltpu.SemaphoreType.DMA,
        ],
    )
    def k(dst_hbm, out_hbm, di_v, ones_v, zw_v, wb_v, deg_sh, sem):
        c = lax.axis_index("c")
        s = lax.axis_index("s")
        wid = s * NC + c
        for j in range(8):
            ones_v[pl.ds(j * 16, 16)] = jnp.ones((16,), jnp.float32)
        for j in range(80):
            zw_v[pl.ds(j * 16, 16)] = jnp.zeros((16,), jnp.float32)
        @pl.when(s < 8)
        def _():
            pltpu.sync_copy(zw_v, deg_sh.at[pl.ds(s * 1280, 1280)])
        plsc.subcore_barrier()

        nblk = jnp.where(wid < TOTB - NW * (BPW - 1), BPW, BPW - 1)
        base_blk = BPW * wid - jnp.maximum(0, wid - (TOTB - NW * (BPW - 1)))

        def body(i, carry):
            @pl.when(i < nblk)
            def _():
                row = (base_blk + i) * IR
                pltpu.sync_copy(dst_hbm.at[pl.ds(row, IR)], di_v)
                for j in range(IR):
                    pltpu.sync_copy(ones_v, deg_sh.at[di_v.at[j]], add=True)
            return carry

        lax.fori_loop(0, BPW, body, 0)
        plsc.subcore_barrier()
        @pl.when(s < 5)
        def _():
            pltpu.sync_copy(deg_sh.at[pl.ds(s * 2000, 2000)], wb_v)
            pltpu.sync_copy(wb_v, out_hbm.at[pl.ds(c * N + s * 2000, 2000)])

    return k(dst2d)


def _sc_aggregate(table, src2d, dst2d, zeros256):
    @functools.partial(
        pl.kernel,
        out_type=jax.ShapeDtypeStruct((NC, N, D), jnp.float32),
        mesh=_mesh(),
        scratch_types=[
            pltpu.VMEM((IR, 128), jnp.int32),
            pltpu.VMEM((IR, 128), jnp.int32),
            pltpu.VMEM((256, D), jnp.float32),
            pltpu.VMEM_SHARED((N_PAD, D), jnp.float32),
            pltpu.SemaphoreType.DMA,
            pltpu.SemaphoreType.DMA,
        ],
    )
    def k(t_hbm, src_hbm, dst_hbm, z_hbm, out_hbm, si_v, di_v, rows_v,
          acc_sh, gsem, ssem):
        c = lax.axis_index("c")
        s = lax.axis_index("s")
        wid = s * NC + c
        pltpu.sync_copy(z_hbm, rows_v)
        zbase = s * 632
        zcps = [pltpu.async_copy(rows_v, acc_sh.at[pl.ds(zbase + j * 256, 256)],
                                 gsem) for j in range(2)]
        zcps.append(pltpu.async_copy(rows_v.at[pl.ds(0, 120)],
                                     acc_sh.at[pl.ds(zbase + 512, 120)], gsem))
        for cp in zcps:
            cp.wait()
        plsc.subcore_barrier()

        nblk = jnp.where(wid < TOTB - NW * (BPW - 1), BPW, BPW - 1)
        base_blk = BPW * wid - jnp.maximum(0, wid - (TOTB - NW * (BPW - 1)))

        def body(blk, carry):
            row = (base_blk + blk) * IR
            pltpu.sync_copy(src_hbm.at[pl.ds(row, IR)], si_v)
            pltpu.sync_copy(dst_hbm.at[pl.ds(row, IR)], di_v)
            NU = IR
            g = [None] * NU
            sd = [None] * NU
            g[0] = pltpu.async_copy(t_hbm.at[si_v.at[0]],
                                    rows_v.at[pl.ds(0, 128)], gsem)
            for j in range(NU):
                g[j].wait()
                if j < NU - 1:
                    if j >= 1:
                        sd[j - 1].wait()
                    g[j + 1] = pltpu.async_copy(
                        t_hbm.at[si_v.at[j + 1]],
                        rows_v.at[pl.ds(((j + 1) % 2) * 128, 128)], gsem)
                sd[j] = pltpu.async_copy(
                    rows_v.at[pl.ds((j % 2) * 128, 128)],
                    acc_sh.at[di_v.at[j]], ssem, add=True)
            sd[NU - 2].wait()
            sd[NU - 1].wait()
            return carry

        def guarded(blk, carry):
            @pl.when(blk < nblk)
            def _():
                body(blk, 0)
            return carry

        lax.fori_loop(0, BPW, guarded, 0)
        plsc.subcore_barrier()
        wbase = s * 632

        def writeout(base_row, sizes):
            offs = []
            o = 0
            for sz in sizes:
                offs.append(o)
                o += sz
            a = [None] * len(sizes)
            b = [None] * len(sizes)
            a[0] = pltpu.async_copy(
                acc_sh.at[pl.ds(base_row + offs[0], sizes[0])],
                rows_v.at[pl.ds(0, sizes[0])], gsem)
            for j, sz in enumerate(sizes):
                h = (j % 2) * 128
                a[j].wait()
                if j + 1 < len(sizes):
                    if j >= 1:
                        b[j - 1].wait()
                    hn = ((j + 1) % 2) * 128
                    a[j + 1] = pltpu.async_copy(
                        acc_sh.at[pl.ds(base_row + offs[j + 1], sizes[j + 1])],
                        rows_v.at[pl.ds(hn, sizes[j + 1])], gsem)
                b[j] = pltpu.async_copy(
                    rows_v.at[pl.ds(h, sz)],
                    out_hbm.at[c, pl.ds(base_row + offs[j], sz)], ssem)
            for j in range(max(0, len(sizes) - 2), len(sizes)):
                b[j].wait()

        @pl.when(s < 15)
        def _():
            writeout(wbase, [128, 128, 128, 128, 120])

        @pl.when(s == 15)
        def _():
            writeout(9480, [128, 128, 128, 128, 8])

    return k(table, src2d, dst2d, zeros256)


def _row_spec(block_rows, cols):
    return pl.BlockSpec((block_rows, cols), lambda i: (i, 0))


def _fixed_spec(shape):
    nd = len(shape)
    return pl.BlockSpec(shape, lambda i: (0,) * nd)


def _tc_mm(x, W1):
    def body(x_ref, w_ref, out_ref):
        out_ref[...] = jnp.dot(x_ref[...], w_ref[...],
                               preferred_element_type=jnp.float32)

    return pl.pallas_call(
        body,
        grid=(GRID,),
        in_specs=[_row_spec(R, D), _fixed_spec((D, D))],
        out_specs=_row_spec(R, D),
        out_shape=jax.ShapeDtypeStruct((N, D), jnp.float32),
    )(x, W1)


def _tc_first(degpair, mm):
    def body(deg_ref, mm_ref, t_ref, dis_ref):
        deg = deg_ref[...]
        dis = lax.rsqrt(deg[:, 0:1] + deg[:, 1:2] + 1.0)
        dis_ref[...] = dis
        t_ref[...] = dis * mm_ref[...]

    return pl.pallas_call(
        body,
        grid=(GRID,),
        in_specs=[_row_spec(R, 2), _row_spec(R, D)],
        out_specs=[_row_spec(R, D), _row_spec(R, 1)],
        out_shape=[jax.ShapeDtypeStruct((N, D), jnp.float32),
                   jax.ShapeDtypeStruct((N, 1), jnp.float32)],
    )(degpair, mm)


def _tc_mid(dis, S, t_prev, b_prev, W_next):
    def body(dis_ref, sa_ref, sb_ref, t_ref, b_ref, w_ref, out_ref):
        dis = dis_ref[...]
        h = dis * (sa_ref[0] + sb_ref[0] + t_ref[...]) + b_ref[...]
        h = jnp.maximum(h, 0.0)
        out_ref[...] = dis * jnp.dot(h, w_ref[...],
                                     preferred_element_type=jnp.float32)

    return pl.pallas_call(
        body,
        grid=(GRID,),
        in_specs=[
            _row_spec(R, 1),
            pl.BlockSpec((1, R, D), lambda i: (0, i, 0)),
            pl.BlockSpec((1, R, D), lambda i: (1, i, 0)),
            _row_spec(R, D),
            _fixed_spec((1, D)),
            _fixed_spec((D, D)),
        ],
        out_specs=_row_spec(R, D),
        out_shape=jax.ShapeDtypeStruct((N, D), jnp.float32),
    )(dis, S, S, t_prev, b_prev, W_next)


def _tc_last(dis, S, t_prev, b_prev):
    def body(dis_ref, sa_ref, sb_ref, t_ref, b_ref, out_ref):
        out_ref[...] = (dis_ref[...] * (sa_ref[0] + sb_ref[0] + t_ref[...])
                        + b_ref[...])

    return pl.pallas_call(
        body,
        grid=(GRID,),
        in_specs=[
            _row_spec(R, 1),
            pl.BlockSpec((1, R, D), lambda i: (0, i, 0)),
            pl.BlockSpec((1, R, D), lambda i: (1, i, 0)),
            _row_spec(R, D),
            _fixed_spec((1, D)),
        ],
        out_specs=_row_spec(R, D),
        out_shape=jax.ShapeDtypeStruct((N, D), jnp.float32),
    )(dis, S, S, t_prev, b_prev)


def kernel(x, edge_index, W1, b1, W2, b2, W3, b3, W4, b4):
    pad = E_PAD - E
    pad_src = jnp.arange(pad, dtype=jnp.int32) % N
    src2d = jnp.concatenate([edge_index[0], pad_src]).reshape(EROWS, 128)
    dump = N + (jnp.arange(pad, dtype=jnp.int32) % (N_PAD - N))
    dst2d = jnp.concatenate([edge_index[1], dump]).reshape(EROWS, 128)
    zeros256 = jnp.zeros((256, D), jnp.float32)
    degp = _sc_degree(dst2d)
    mm1 = _tc_mm(x, W1)
    degpair = degp.reshape(NC, N).T
    t1, dis = _tc_first(degpair, mm1)
    s1 = _sc_aggregate(t1, src2d, dst2d, zeros256)
    t2 = _tc_mid(dis, s1, t1, b1.reshape(1, D), W2)
    s2 = _sc_aggregate(t2, src2d, dst2d, zeros256)
    t3 = _tc_mid(dis, s2, t2, b2.reshape(1, D), W3)
    s3 = _sc_aggregate(t3, src2d, dst2d, zeros256)
    t4 = _tc_mid(dis, s3, t3, b3.reshape(1, D), W4)
    s4 = _sc_aggregate(t4, src2d, dst2d, zeros256)
    return _tc_last(dis, s4, t4, b4.reshape(1, D))
